# baseline (speedup 1.0000x reference)
import functools

import jax
import jax.numpy as jnp
from jax import lax
from jax.experimental import pallas as pl
from jax.experimental.pallas import tpu as pltpu
from jax.experimental.pallas import tpu_sc as plsc

NUM_GRAPHS = 2048
NUM_NODES = 10000
FEAT = 512

NC = 2
NS = 16
NW = NC * NS
SEG_PER_TILE = NUM_GRAPHS // NW
WIN = 64
SAMP = 256
NPAD = 10240


def _sc_pool(x, batch_i32, zsums, zcnt):
    mesh = plsc.VectorSubcoreMesh(core_axis_name="c", subcore_axis_name="s")

    @functools.partial(
        pl.kernel,
        out_type=[
            jax.ShapeDtypeStruct((NUM_GRAPHS, FEAT), jnp.float32),
            jax.ShapeDtypeStruct((NUM_GRAPHS, 16), jnp.float32),
        ],
        mesh=mesh,
        compiler_params=pltpu.CompilerParams(needs_layout_passes=False),
        scratch_types=[
            pltpu.VMEM((NPAD,), jnp.int32),
            pltpu.VMEM((WIN, FEAT), jnp.float32),
            pltpu.VMEM((WIN, FEAT), jnp.float32),
            pltpu.VMEM((SEG_PER_TILE + 1, FEAT), jnp.float32),
            pltpu.VMEM((SEG_PER_TILE + 1, 16), jnp.float32),
            pltpu.SemaphoreType.DMA,
            pltpu.SemaphoreType.DMA,
        ],
    )
    def body(x_hbm, b_hbm, zs_hbm, zc_hbm, sums_out, cnt_out,
             idbuf, xb0, xb1, acc, cnt, sem0, sem1):
        cid = lax.axis_index("c")
        sid = lax.axis_index("s")
        wid = cid * NS + sid
        seg_lo = wid * SEG_PER_TILE
        iota16 = lax.iota(jnp.int32, 16)
        e0i = jnp.where(iota16 == 0, 1, 0)
        z16 = jnp.zeros((16,), jnp.int32)
        e0_16 = jnp.where(iota16 == 0, 1.0, 0.0).astype(jnp.float32)
        sent16 = jnp.full((16,), NUM_GRAPHS, jnp.int32)

        zs_cp = pltpu.make_async_copy(zs_hbm, acc.at[pl.ds(0, SEG_PER_TILE)],
                                      sem0)
        zc_cp = pltpu.make_async_copy(zc_hbm, cnt.at[pl.ds(0, SEG_PER_TILE)],
                                      sem1)
        zs_cp.start()
        zc_cp.start()
        pltpu.sync_copy(b_hbm, idbuf.at[pl.ds(0, NUM_NODES)])
        for t in range((NPAD - NUM_NODES) // 16):
            idbuf[pl.ds(NUM_NODES + t * 16, 16)] = sent16

        thr1 = seg_lo
        thr2 = seg_lo + SEG_PER_TILE

        def l1_body(j, carry):
            a1, a2 = carry
            v = idbuf[pl.ds(j * SAMP, 16)]
            a1 = a1 + jnp.where(v < thr1, e0i, 0)
            a2 = a2 + jnp.where(v < thr2, e0i, 0)
            return a1, a2

        a1, a2 = lax.fori_loop(0, NPAD // SAMP, l1_body, (z16, z16))
        b1 = jnp.maximum(jnp.sum(a1) - 1, 0) * SAMP
        b2 = jnp.maximum(jnp.sum(a2) - 1, 0) * SAMP

        def l2_body(i, carry):
            c1, c2 = carry
            v1 = idbuf[pl.ds(b1 + i * 16, 16)]
            v2 = idbuf[pl.ds(b2 + i * 16, 16)]
            c1 = c1 + jnp.where(v1 < thr1, 1, 0)
            c2 = c2 + jnp.where(v2 < thr2, 1, 0)
            return c1, c2

        c1, c2 = lax.fori_loop(0, SAMP // 16, l2_body, (z16, z16))
        lo = b1 + jnp.sum(c1)
        hi = b2 + jnp.sum(c2)

        w0 = (lo // WIN) * WIN
        nwin = (hi - w0 + (WIN - 1)) // WIN

        def win_start(w, buf, sem):
            s = jnp.minimum(w0 + WIN * w, NUM_NODES - WIN)
            pltpu.make_async_copy(x_hbm.at[pl.ds(s, WIN)], buf, sem).start()

        def win_wait(buf, sem):
            pltpu.make_async_copy(x_hbm.at[pl.ds(0, WIN)], buf, sem).wait()

        def process(w, buf):
            s_true = w0 + WIN * w
            s = jnp.minimum(s_true, NUM_NODES - WIN)
            lo2 = jnp.maximum(lo, s_true)

            def grp_body(k, carry2):
                g0 = s + k * 16
                idv = idbuf[pl.ds(g0, 16)]
                for j in range(16):
                    g = g0 + j
                    valid = (g >= lo2) & (g < hi)
                    sid_t = jnp.where(valid, idv[j] - seg_lo, SEG_PER_TILE)
                    rloc = k * 16 + j
                    plsc.addupdate(cnt.at[sid_t], e0_16)
                    for c in range(FEAT // 16):
                        plsc.addupdate(acc.at[sid_t, pl.ds(c * 16, 16)],
                                       buf[rloc, pl.ds(c * 16, 16)])
                return carry2

            lax.fori_loop(0, WIN // 16, grp_body, 0)

        zs_cp.wait()
        zc_cp.wait()

        @pl.when(nwin > 0)
        def _():
            win_start(0, xb0, sem0)

        def outer(h, carry):
            w = h * 2

            @pl.when(w + 1 < nwin)
            def _():
                win_start(w + 1, xb1, sem1)

            win_wait(xb0, sem0)
            process(w, xb0)

            @pl.when(w + 2 < nwin)
            def _():
                win_start(w + 2, xb0, sem0)

            @pl.when(w + 1 < nwin)
            def _():
                win_wait(xb1, sem1)
                process(w + 1, xb1)

            return carry

        lax.fori_loop(0, (nwin + 1) // 2, outer, 0)

        pltpu.sync_copy(acc.at[pl.ds(0, SEG_PER_TILE)],
                        sums_out.at[pl.ds(seg_lo, SEG_PER_TILE)])
        pltpu.sync_copy(cnt.at[pl.ds(0, SEG_PER_TILE)],
                        cnt_out.at[pl.ds(seg_lo, SEG_PER_TILE)])

    return body(x, batch_i32, zsums, zcnt)


def _tc_body(s_ref, c_ref, wxg_ref, b_ref, wlin_ref, o_ref):
    denom = jnp.maximum(jnp.sum(c_ref[...], axis=1, keepdims=True), 1.0)
    m = s_ref[...] / denom
    h = lax.dot_general(m, wxg_ref[...], (((1,), (1,)), ((), ())),
                        preferred_element_type=jnp.float32) + b_ref[...]
    r = jnp.maximum(h, 0.0)
    o_ref[...] = h + lax.dot_general(r, wlin_ref[...], (((1,), (1,)), ((), ())),
                                     preferred_element_type=jnp.float32)


def _tc_dense(sums, counts, W_xg, b_xg2, W_lin):
    blk = 256
    grid = NUM_GRAPHS // blk
    return pl.pallas_call(
        _tc_body,
        grid=(grid,),
        in_specs=[
            pl.BlockSpec((blk, FEAT), lambda i: (i, 0)),
            pl.BlockSpec((blk, 16), lambda i: (i, 0)),
            pl.BlockSpec((FEAT, FEAT), lambda i: (0, 0)),
            pl.BlockSpec((1, FEAT), lambda i: (0, 0)),
            pl.BlockSpec((FEAT, FEAT), lambda i: (0, 0)),
        ],
        out_specs=pl.BlockSpec((blk, FEAT), lambda i: (i, 0)),
        out_shape=jax.ShapeDtypeStruct((NUM_GRAPHS, FEAT), jnp.float32),
    )(sums, counts, W_xg, b_xg2, W_lin)


@jax.jit
def kernel(x, batch, W_xg, b_xg, W_lin):
    batch_i32 = batch.astype(jnp.int32)
    zsums = jnp.zeros((SEG_PER_TILE, FEAT), jnp.float32)
    zcnt = jnp.zeros((SEG_PER_TILE, 16), jnp.float32)
    sums, counts = _sc_pool(x, batch_i32, zsums, zcnt)
    return _tc_dense(sums, counts, W_xg, b_xg.reshape(1, FEAT), W_lin)


# speedup vs baseline: 1.1413x; 1.1413x over previous
import functools

import jax
import jax.numpy as jnp
from jax import lax
from jax.experimental import pallas as pl
from jax.experimental.pallas import tpu as pltpu
from jax.experimental.pallas import tpu_sc as plsc

NUM_GRAPHS = 2048
NUM_NODES = 10000
FEAT = 512

NC = 2
NS = 16
NW = NC * NS
SEG_PER_TILE = NUM_GRAPHS // NW
WIN = 64
SAMP = 256
NPAD = 10240


def _sc_pool(x, batch_i32, zsums, zcnt):
    mesh = plsc.VectorSubcoreMesh(core_axis_name="c", subcore_axis_name="s")

    @functools.partial(
        pl.kernel,
        out_type=[
            jax.ShapeDtypeStruct((NUM_GRAPHS, FEAT), jnp.float32),
            jax.ShapeDtypeStruct((NUM_GRAPHS, 16), jnp.float32),
        ],
        mesh=mesh,
        compiler_params=pltpu.CompilerParams(needs_layout_passes=False),
        scratch_types=[
            pltpu.VMEM((NPAD,), jnp.int32),
            pltpu.VMEM((WIN, FEAT), jnp.float32),
            pltpu.VMEM((WIN, FEAT), jnp.float32),
            pltpu.VMEM((SEG_PER_TILE + 1, FEAT), jnp.float32),
            pltpu.VMEM((SEG_PER_TILE + 1, 16), jnp.float32),
            pltpu.SemaphoreType.DMA,
            pltpu.SemaphoreType.DMA,
        ],
    )
    def body(x_hbm, b_hbm, zs_hbm, zc_hbm, sums_out, cnt_out,
             idbuf, xb0, xb1, acc, cnt, sem0, sem1):
        cid = lax.axis_index("c")
        sid = lax.axis_index("s")
        wid = cid * NS + sid
        seg_lo = wid * SEG_PER_TILE
        iota16 = lax.iota(jnp.int32, 16)
        e0i = jnp.where(iota16 == 0, 1, 0)
        z16 = jnp.zeros((16,), jnp.int32)
        e0_16 = jnp.where(iota16 == 0, 1.0, 0.0).astype(jnp.float32)
        sent16 = jnp.full((16,), NUM_GRAPHS, jnp.int32)

        zs_cp = pltpu.make_async_copy(zs_hbm, acc.at[pl.ds(0, SEG_PER_TILE)],
                                      sem0)
        zc_cp = pltpu.make_async_copy(zc_hbm, cnt.at[pl.ds(0, SEG_PER_TILE)],
                                      sem1)
        zs_cp.start()
        zc_cp.start()
        pltpu.sync_copy(b_hbm, idbuf.at[pl.ds(0, NUM_NODES)])
        for t in range((NPAD - NUM_NODES) // 16):
            idbuf[pl.ds(NUM_NODES + t * 16, 16)] = sent16

        thr1 = seg_lo
        thr2 = seg_lo + SEG_PER_TILE

        def l1_body(j, carry):
            a1, a2 = carry
            v = idbuf[pl.ds(j * SAMP, 16)]
            a1 = a1 + jnp.where(v < thr1, e0i, 0)
            a2 = a2 + jnp.where(v < thr2, e0i, 0)
            return a1, a2

        a1, a2 = lax.fori_loop(0, NPAD // SAMP, l1_body, (z16, z16))
        b1 = jnp.maximum(jnp.sum(a1) - 1, 0) * SAMP
        b2 = jnp.maximum(jnp.sum(a2) - 1, 0) * SAMP

        def l2_body(i, carry):
            c1, c2 = carry
            v1 = idbuf[pl.ds(b1 + i * 16, 16)]
            v2 = idbuf[pl.ds(b2 + i * 16, 16)]
            c1 = c1 + jnp.where(v1 < thr1, 1, 0)
            c2 = c2 + jnp.where(v2 < thr2, 1, 0)
            return c1, c2

        c1, c2 = lax.fori_loop(0, SAMP // 16, l2_body, (z16, z16))
        lo = b1 + jnp.sum(c1)
        hi = b2 + jnp.sum(c2)

        w0 = (lo // WIN) * WIN
        nwin = (hi - w0 + (WIN - 1)) // WIN

        def win_start(w, buf, sem):
            s = jnp.minimum(w0 + WIN * w, NUM_NODES - WIN)
            pltpu.make_async_copy(x_hbm.at[pl.ds(s, WIN)], buf, sem).start()

        def win_wait(buf, sem):
            pltpu.make_async_copy(x_hbm.at[pl.ds(0, WIN)], buf, sem).wait()

        def process(w, buf):
            s_true = w0 + WIN * w
            s = jnp.minimum(s_true, NUM_NODES - WIN)
            lo2 = jnp.maximum(lo, s_true)

            def grp_body(k, carry2):
                g0 = s + k * 16
                idv = idbuf[pl.ds(g0, 16)]
                for j in range(16):
                    g = g0 + j
                    valid = (g >= lo2) & (g < hi)
                    sid_t = jnp.where(valid, idv[j] - seg_lo, SEG_PER_TILE)
                    rloc = k * 16 + j
                    plsc.addupdate(cnt.at[sid_t], e0_16)
                    for c in range(FEAT // 16):
                        plsc.addupdate(acc.at[sid_t, pl.ds(c * 16, 16)],
                                       buf[rloc, pl.ds(c * 16, 16)])
                return carry2

            lax.fori_loop(0, WIN // 16, grp_body, 0)

        zs_cp.wait()
        zc_cp.wait()

        def outer(w, carry):
            s = jnp.minimum(w0 + WIN * w, NUM_NODES - WIN)
            pltpu.sync_copy(x_hbm.at[pl.ds(s, WIN)], xb0)
            process(w, xb0)
            return carry

        lax.fori_loop(0, nwin, outer, 0)

        pltpu.sync_copy(acc.at[pl.ds(0, SEG_PER_TILE)],
                        sums_out.at[pl.ds(seg_lo, SEG_PER_TILE)])
        pltpu.sync_copy(cnt.at[pl.ds(0, SEG_PER_TILE)],
                        cnt_out.at[pl.ds(seg_lo, SEG_PER_TILE)])

    return body(x, batch_i32, zsums, zcnt)


def _tc_body(s_ref, c_ref, wxg_ref, b_ref, wlin_ref, o_ref):
    denom = jnp.maximum(jnp.sum(c_ref[...], axis=1, keepdims=True), 1.0)
    m = s_ref[...] / denom
    h = lax.dot_general(m, wxg_ref[...], (((1,), (1,)), ((), ())),
                        preferred_element_type=jnp.float32) + b_ref[...]
    r = jnp.maximum(h, 0.0)
    o_ref[...] = h + lax.dot_general(r, wlin_ref[...], (((1,), (1,)), ((), ())),
                                     preferred_element_type=jnp.float32)


def _tc_dense(sums, counts, W_xg, b_xg2, W_lin):
    blk = 256
    grid = NUM_GRAPHS // blk
    return pl.pallas_call(
        _tc_body,
        grid=(grid,),
        in_specs=[
            pl.BlockSpec((blk, FEAT), lambda i: (i, 0)),
            pl.BlockSpec((blk, 16), lambda i: (i, 0)),
            pl.BlockSpec((FEAT, FEAT), lambda i: (0, 0)),
            pl.BlockSpec((1, FEAT), lambda i: (0, 0)),
            pl.BlockSpec((FEAT, FEAT), lambda i: (0, 0)),
        ],
        out_specs=pl.BlockSpec((blk, FEAT), lambda i: (i, 0)),
        out_shape=jax.ShapeDtypeStruct((NUM_GRAPHS, FEAT), jnp.float32),
    )(sums, counts, W_xg, b_xg2, W_lin)


@jax.jit
def kernel(x, batch, W_xg, b_xg, W_lin):
    batch_i32 = batch.astype(jnp.int32)
    zsums = jnp.zeros((SEG_PER_TILE, FEAT), jnp.float32)
    zcnt = jnp.zeros((SEG_PER_TILE, 16), jnp.float32)
    sums, counts = _sc_pool(x, batch_i32, zsums, zcnt)
    return _tc_dense(sums, counts, W_xg, b_xg.reshape(1, FEAT), W_lin)


# speedup vs baseline: 1.6960x; 1.4860x over previous
import functools

import jax
import jax.numpy as jnp
from jax import lax
from jax.experimental import pallas as pl
from jax.experimental.pallas import tpu as pltpu
from jax.experimental.pallas import tpu_sc as plsc

NUM_GRAPHS = 2048
NUM_NODES = 10000
FEAT = 512

NC = 2
NS = 16
NW = NC * NS
SEG_PER_TILE = NUM_GRAPHS // NW
WIN = 64
SAMP = 256
NPAD = 10240


def _sc_pool(x, batch_i32, zsums, zcnt):
    mesh = plsc.VectorSubcoreMesh(core_axis_name="c", subcore_axis_name="s")

    @functools.partial(
        pl.kernel,
        out_type=[
            jax.ShapeDtypeStruct((NUM_GRAPHS, FEAT), jnp.float32),
            jax.ShapeDtypeStruct((NUM_GRAPHS, 16), jnp.float32),
        ],
        mesh=mesh,
        compiler_params=pltpu.CompilerParams(needs_layout_passes=False),
        scratch_types=[
            pltpu.VMEM((NPAD,), jnp.int32),
            pltpu.VMEM((WIN, FEAT), jnp.float32),
            pltpu.VMEM((WIN, FEAT), jnp.float32),
            pltpu.VMEM((SEG_PER_TILE + 1, FEAT), jnp.float32),
            pltpu.VMEM((SEG_PER_TILE + 1, 16), jnp.float32),
            pltpu.SemaphoreType.DMA,
            pltpu.SemaphoreType.DMA,
        ],
    )
    def body(x_hbm, b_hbm, zs_hbm, zc_hbm, sums_out, cnt_out,
             idbuf, xb0, xb1, acc, cnt, sem0, sem1):
        cid = lax.axis_index("c")
        sid = lax.axis_index("s")
        wid = cid * NS + sid
        seg_lo = wid * SEG_PER_TILE
        iota16 = lax.iota(jnp.int32, 16)
        e0i = jnp.where(iota16 == 0, 1, 0)
        z16 = jnp.zeros((16,), jnp.int32)
        e0_16 = jnp.where(iota16 == 0, 1.0, 0.0).astype(jnp.float32)
        sent16 = jnp.full((16,), NUM_GRAPHS, jnp.int32)

        zs_cp = pltpu.make_async_copy(zs_hbm, acc.at[pl.ds(0, SEG_PER_TILE)],
                                      sem0)
        zc_cp = pltpu.make_async_copy(zc_hbm, cnt.at[pl.ds(0, SEG_PER_TILE)],
                                      sem1)
        zs_cp.start()
        zc_cp.start()
        pltpu.sync_copy(b_hbm, idbuf.at[pl.ds(0, NUM_NODES)])
        for t in range((NPAD - NUM_NODES) // 16):
            idbuf[pl.ds(NUM_NODES + t * 16, 16)] = sent16

        thr1 = seg_lo
        thr2 = seg_lo + SEG_PER_TILE

        def l1_body(j, carry):
            a1, a2 = carry
            v = idbuf[pl.ds(j * SAMP, 16)]
            a1 = a1 + jnp.where(v < thr1, e0i, 0)
            a2 = a2 + jnp.where(v < thr2, e0i, 0)
            return a1, a2

        a1, a2 = lax.fori_loop(0, NPAD // SAMP, l1_body, (z16, z16))
        b1 = jnp.maximum(jnp.sum(a1) - 1, 0) * SAMP
        b2 = jnp.maximum(jnp.sum(a2) - 1, 0) * SAMP

        def l2_body(i, carry):
            c1, c2 = carry
            v1 = idbuf[pl.ds(b1 + i * 16, 16)]
            v2 = idbuf[pl.ds(b2 + i * 16, 16)]
            c1 = c1 + jnp.where(v1 < thr1, 1, 0)
            c2 = c2 + jnp.where(v2 < thr2, 1, 0)
            return c1, c2

        c1, c2 = lax.fori_loop(0, SAMP // 16, l2_body, (z16, z16))
        lo = b1 + jnp.sum(c1)
        hi = b2 + jnp.sum(c2)

        w0 = (lo // WIN) * WIN
        nwin = (hi - w0 + (WIN - 1)) // WIN

        def win_start(w, buf, sem):
            s = jnp.minimum(w0 + WIN * w, NUM_NODES - WIN)
            pltpu.make_async_copy(x_hbm.at[pl.ds(s, WIN)], buf, sem).start()

        def win_wait(buf, sem):
            pltpu.make_async_copy(x_hbm.at[pl.ds(0, WIN)], buf, sem).wait()

        def process(w, buf):
            s_true = w0 + WIN * w
            s = jnp.minimum(s_true, NUM_NODES - WIN)
            lo2 = jnp.maximum(lo, s_true)

            def grp_body(k, carry2):
                g0 = s + k * 16
                idv = idbuf[pl.ds(g0, 16)]
                gv = g0 + iota16
                validv = (gv >= lo2) & (gv < hi)
                sidv = jnp.where(validv, idv - seg_lo, SEG_PER_TILE)
                for j in range(16):
                    sid_t = sidv[j]
                    rloc = k * 16 + j
                    vals = [buf[rloc, pl.ds(c * 16, 16)]
                            for c in range(FEAT // 16)]
                    plsc.addupdate(cnt.at[sid_t], e0_16)
                    for c in range(FEAT // 16):
                        plsc.addupdate(acc.at[sid_t, pl.ds(c * 16, 16)],
                                       vals[c])
                return carry2

            lax.fori_loop(0, WIN // 16, grp_body, 0)

        zs_cp.wait()
        zc_cp.wait()

        def outer(w, carry):
            s = jnp.minimum(w0 + WIN * w, NUM_NODES - WIN)
            pltpu.sync_copy(x_hbm.at[pl.ds(s, WIN)], xb0)
            process(w, xb0)
            return carry

        lax.fori_loop(0, nwin, outer, 0)

        pltpu.sync_copy(acc.at[pl.ds(0, SEG_PER_TILE)],
                        sums_out.at[pl.ds(seg_lo, SEG_PER_TILE)])
        pltpu.sync_copy(cnt.at[pl.ds(0, SEG_PER_TILE)],
                        cnt_out.at[pl.ds(seg_lo, SEG_PER_TILE)])

    return body(x, batch_i32, zsums, zcnt)


def _tc_body(s_ref, c_ref, wxg_ref, b_ref, wlin_ref, o_ref):
    denom = jnp.maximum(jnp.sum(c_ref[...], axis=1, keepdims=True), 1.0)
    m = s_ref[...] / denom
    h = lax.dot_general(m, wxg_ref[...], (((1,), (1,)), ((), ())),
                        preferred_element_type=jnp.float32) + b_ref[...]
    r = jnp.maximum(h, 0.0)
    o_ref[...] = h + lax.dot_general(r, wlin_ref[...], (((1,), (1,)), ((), ())),
                                     preferred_element_type=jnp.float32)


def _tc_dense(sums, counts, W_xg, b_xg2, W_lin):
    blk = 256
    grid = NUM_GRAPHS // blk
    return pl.pallas_call(
        _tc_body,
        grid=(grid,),
        in_specs=[
            pl.BlockSpec((blk, FEAT), lambda i: (i, 0)),
            pl.BlockSpec((blk, 16), lambda i: (i, 0)),
            pl.BlockSpec((FEAT, FEAT), lambda i: (0, 0)),
            pl.BlockSpec((1, FEAT), lambda i: (0, 0)),
            pl.BlockSpec((FEAT, FEAT), lambda i: (0, 0)),
        ],
        out_specs=pl.BlockSpec((blk, FEAT), lambda i: (i, 0)),
        out_shape=jax.ShapeDtypeStruct((NUM_GRAPHS, FEAT), jnp.float32),
    )(sums, counts, W_xg, b_xg2, W_lin)


@jax.jit
def kernel(x, batch, W_xg, b_xg, W_lin):
    batch_i32 = batch.astype(jnp.int32)
    zsums = jnp.zeros((SEG_PER_TILE, FEAT), jnp.float32)
    zcnt = jnp.zeros((SEG_PER_TILE, 16), jnp.float32)
    sums, counts = _sc_pool(x, batch_i32, zsums, zcnt)
    return _tc_dense(sums, counts, W_xg, b_xg.reshape(1, FEAT), W_lin)
